# baseline (device time: 104143 ns/iter reference)
import jax
import jax.numpy as jnp
from jax import lax
from jax.experimental import pallas as pl
from jax.experimental.pallas import tpu as pltpu

N_DEV = 16
CLIP = 5.0


def kernel(A, B):
    m_per, k = A.shape
    _, n = B.shape

    def body(a_ref, b_ref, out_ref, gathered, send_sems, recv_sems):
        my = lax.axis_index("i")
        q = lax.rem(my, 4)
        z = lax.div(my, 4)
        b = lax.rem(z, 2) * 2 + lax.div(z, 2)
        my_slot = q * 4 + b

        def pos_of(slot):
            qq = lax.div(slot, 4)
            bb = lax.rem(slot, 4)
            zz = lax.rem(bb, 2) * 2 + lax.div(bb, 2)
            return 4 * zz + qq

        pA = 4 * (z ^ 2) + q
        pB = 4 * (z ^ 1) + q
        pC = 4 * z + (q ^ 1)
        pD = 4 * z + (q ^ 3)

        a_q = jnp.clip(
            jnp.round(a_ref[...] * (127.0 / CLIP)), -127.0, 127.0
        ).astype(jnp.int8)
        gathered[my_slot] = a_q
        b_scaled = (b_ref[...] * (CLIP / 127.0)).astype(jnp.bfloat16)

        barrier_sem = pltpu.get_barrier_semaphore()
        for p in (pA, pB, pC, pD):
            pl.semaphore_signal(
                barrier_sem, inc=1,
                device_id=(p,), device_id_type=pl.DeviceIdType.MESH,
            )
        pl.semaphore_wait(barrier_sem, 4)

        def dot_store(origin, chunk):
            out_ref[pl.ds(origin * m_per, m_per), :] = jnp.dot(
                chunk.astype(jnp.bfloat16), b_scaled,
                preferred_element_type=jnp.float32,
            )

        def exchange(step, send_start, size, recv_start, partner):
            snd = pltpu.make_async_remote_copy(
                src_ref=gathered.at[pl.ds(send_start, size)],
                dst_ref=gathered.at[pl.ds(send_start, size)],
                send_sem=send_sems.at[step],
                recv_sem=recv_sems.at[step],
                device_id=(partner,), device_id_type=pl.DeviceIdType.MESH,
            )
            snd.start()
            rcv = pltpu.make_async_remote_copy(
                src_ref=gathered.at[pl.ds(send_start, size)],
                dst_ref=gathered.at[pl.ds(recv_start, size)],
                send_sem=send_sems.at[step],
                recv_sem=recv_sems.at[step],
                device_id=(partner,), device_id_type=pl.DeviceIdType.MESH,
            )
            return snd, rcv

        run2 = q * 4 + lax.div(b, 2) * 2
        run4 = q * 4
        run8 = 8 * lax.div(q, 2)

        recvA = q * 4 + (b ^ 1)
        recvB = q * 4 + (lax.div(b, 2) ^ 1) * 2
        recvC = (q ^ 1) * 4
        recvD = 8 * (lax.div(q, 2) ^ 1)

        sA, rA = exchange(0, my_slot, 1, recvA, pA)
        dot_store(my, a_q)
        rA.wait_recv()

        sB, rB = exchange(1, run2, 2, recvB, pB)
        dot_store(pos_of(recvA), gathered[recvA])
        rB.wait_recv()

        sC, rC = exchange(2, run4, 4, recvC, pC)
        for j in range(2):
            slot = recvB + j
            dot_store(pos_of(slot), gathered[slot])
        rC.wait_recv()

        sD1, rD1 = exchange(3, run8, 4, recvD, pD)
        sD2, rD2 = exchange(4, run8 + 4, 4, recvD + 4, pD)
        for j in range(4):
            slot = recvC + j
            dot_store(pos_of(slot), gathered[slot])
        rD1.wait_recv()
        for j in range(4):
            slot = recvD + j
            dot_store(pos_of(slot), gathered[slot])
        rD2.wait_recv()
        for j in range(4):
            slot = recvD + 4 + j
            dot_store(pos_of(slot), gathered[slot])

        for snd in (sA, sB, sC, sD1, sD2):
            snd.wait_send()

    return pl.pallas_call(
        body,
        out_shape=jax.ShapeDtypeStruct((N_DEV * m_per, n), jnp.float32),
        in_specs=[
            pl.BlockSpec(memory_space=pltpu.VMEM),
            pl.BlockSpec(memory_space=pltpu.VMEM),
        ],
        out_specs=pl.BlockSpec(memory_space=pltpu.VMEM),
        scratch_shapes=[
            pltpu.VMEM((N_DEV, m_per, k), jnp.int8),
            pltpu.SemaphoreType.DMA((5,)),
            pltpu.SemaphoreType.DMA((5,)),
        ],
        compiler_params=pltpu.CompilerParams(
            collective_id=0, vmem_limit_bytes=100 * 1024 * 1024
        ),
    )(A, B)
